# Initial kernel scaffold; baseline (speedup 1.0000x reference)
import jax
import jax.numpy as jnp
from jax.experimental import pallas as pl


def kernel(x, edge_index, W1, b1, gamma, beta, W2, b2):
    raise NotImplementedError("write your pallas kernel here")


# speedup vs baseline: 9.3656x; 9.3656x over previous
import functools

import jax
import jax.numpy as jnp
from jax import lax
from jax.experimental import pallas as pl
from jax.experimental.pallas import tpu as pltpu
from jax.experimental.pallas import tpu_sc as plsc

N = 10000
NPAD = 10240
E = 320000
EPAD = 327680
CH = 128
NC, NS = 2, 16
NW = NC * NS
PER_W = EPAD // NW
NCHUNK = PER_W // CH
RPT = NPAD // NS
BR = 512
GRID = NPAD // BR
IN_CH = 128
LAT = 64
BN_EPS = 1e-5

_mesh = plsc.VectorSubcoreMesh(
    core_axis_name="c", subcore_axis_name="s", num_cores=NC, num_subcores=NS
)


@functools.partial(
    pl.kernel,
    out_type=jax.ShapeDtypeStruct((NC, NPAD, 16), jnp.float32),
    mesh=_mesh,
    scratch_types=[
        pltpu.VMEM((CH,), jnp.int32),
        pltpu.VMEM((CH, 16), jnp.float32),
        pltpu.VMEM_SHARED((NPAD, 16), jnp.float32),
    ],
    compiler_params=pltpu.CompilerParams(use_tc_tiling_on_sc=False),
)
def _sc_degree(dst_hbm, out_hbm, didx, buf, accum):
    cid = lax.axis_index("c")
    sid = lax.axis_index("s")
    wid = sid * NC + cid
    r0 = sid * RPT

    def zrow(i, _):
        buf[i, :] = jnp.zeros((16,), jnp.float32)
        return 0

    lax.fori_loop(0, CH, zrow, 0)

    def zc(k, _):
        pltpu.sync_copy(buf, accum.at[pl.ds(r0 + k * CH, CH)])
        return 0

    lax.fori_loop(0, RPT // CH, zc, 0)
    plsc.subcore_barrier()

    def orow(i, _):
        buf[i, :] = jnp.ones((16,), jnp.float32)
        return 0

    lax.fori_loop(0, CH, orow, 0)
    base = wid * PER_W

    def body(j, _):
        pltpu.sync_copy(dst_hbm.at[pl.ds(base + j * CH, CH)], didx)
        pltpu.sync_copy(buf, accum.at[didx], add=True)
        return 0

    lax.fori_loop(0, NCHUNK, body, 0)
    plsc.subcore_barrier()

    def ro(k, _):
        rr = r0 + k * CH
        pltpu.sync_copy(accum.at[pl.ds(rr, CH)], buf)
        pltpu.sync_copy(buf, out_hbm.at[cid, pl.ds(rr, CH)])
        return 0

    lax.fori_loop(0, RPT // CH, ro, 0)


def _make_sc_msg(D):
    @functools.partial(
        pl.kernel,
        out_type=jax.ShapeDtypeStruct((NC, NPAD, D), jnp.float32),
        mesh=_mesh,
        scratch_types=[
            pltpu.VMEM((CH,), jnp.int32),
            pltpu.VMEM((CH,), jnp.int32),
            pltpu.VMEM((CH, D), jnp.float32),
            pltpu.VMEM_SHARED((NPAD, D), jnp.float32),
            pltpu.SemaphoreType.DMA,
        ],
        compiler_params=pltpu.CompilerParams(use_tc_tiling_on_sc=False),
    )
    def msg(tab_hbm, src_hbm, dst_hbm, out_hbm, sidx, didx, rows, accum, sem):
        cid = lax.axis_index("c")
        sid = lax.axis_index("s")
        wid = sid * NC + cid
        r0 = sid * RPT

        def zrow(i, _):
            def zcol(jj, _):
                rows[i, pl.ds(jj * 16, 16)] = jnp.zeros((16,), jnp.float32)
                return 0

            lax.fori_loop(0, D // 16, zcol, 0)
            return 0

        lax.fori_loop(0, CH, zrow, 0)

        def zc(k, _):
            pltpu.sync_copy(rows, accum.at[pl.ds(r0 + k * CH, CH)])
            return 0

        lax.fori_loop(0, RPT // CH, zc, 0)
        plsc.subcore_barrier()

        base = wid * PER_W

        def body(j, _):
            off = base + j * CH
            pltpu.sync_copy(src_hbm.at[pl.ds(off, CH)], sidx)
            pltpu.sync_copy(dst_hbm.at[pl.ds(off, CH)], didx)
            pltpu.async_copy(tab_hbm.at[sidx], rows, sem).wait()
            pltpu.sync_copy(rows, accum.at[didx], add=True)
            return 0

        lax.fori_loop(0, NCHUNK, body, 0)
        plsc.subcore_barrier()

        def ro(k, _):
            rr = r0 + k * CH
            pltpu.sync_copy(accum.at[pl.ds(rr, CH)], rows)
            pltpu.sync_copy(rows, out_hbm.at[cid, pl.ds(rr, CH)])
            return 0

        lax.fori_loop(0, RPT // CH, ro, 0)

    return msg


_sc_msg128 = _make_sc_msg(IN_CH)
_sc_msg64 = _make_sc_msg(LAT)


def _tc1_body(xp_ref, w1_ref, degp_ref, scaled_ref, dinvb_ref):
    i = pl.program_id(0)
    dp = degp_ref[...]
    deg = dp[0, :, 0:1] + dp[1, :, 0:1] + 1.0
    dinv = lax.rsqrt(deg)
    rows = lax.broadcasted_iota(jnp.int32, (BR, 1), 0) + i * BR
    dinv = jnp.where(rows < N, dinv, 0.0)
    xw = jnp.dot(xp_ref[...], w1_ref[...], preferred_element_type=jnp.float32)
    scaled_ref[...] = xw * dinv
    dinvb_ref[...] = jnp.broadcast_to(dinv, (BR, IN_CH))


def _tc2_body(p_ref, s1_ref, dinvb_ref, b1_ref, out1_ref, stats_ref):
    i = pl.program_id(0)
    p = p_ref[...]
    o = dinvb_ref[...] * (p[0] + p[1] + s1_ref[...]) + b1_ref[...]
    out1_ref[...] = o
    rows = lax.broadcasted_iota(jnp.int32, (BR, 1), 0) + i * BR
    om = jnp.where(rows < N, o, 0.0)

    @pl.when(i == 0)
    def _():
        stats_ref[...] = jnp.zeros_like(stats_ref)

    stats_ref[0:1, :] += jnp.sum(om, axis=0, keepdims=True)
    stats_ref[1:2, :] += jnp.sum(om * om, axis=0, keepdims=True)


def _tc3_body(out1_ref, stats_ref, g_ref, bt_ref, w2_ref, dinvb_ref, s2_ref):
    st = stats_ref[...]
    mean = st[0:1, :] * (1.0 / N)
    var = st[1:2, :] * (1.0 / N) - mean * mean
    h = g_ref[...] * (out1_ref[...] - mean) * lax.rsqrt(var + BN_EPS) + bt_ref[...]
    h = jnp.maximum(h, 0.0)
    hw = jnp.dot(h, w2_ref[...], preferred_element_type=jnp.float32)
    s2_ref[...] = hw * dinvb_ref[:, :LAT]


def _tc4_body(p_ref, s2_ref, dinvb_ref, b2_ref, z_ref):
    p = p_ref[...]
    z_ref[...] = dinvb_ref[:, :LAT] * (p[0] + p[1] + s2_ref[...]) + b2_ref[...]


def _row_spec(d):
    return pl.BlockSpec((BR, d), lambda i: (i, 0))


_tc1 = pl.pallas_call(
    _tc1_body,
    grid=(GRID,),
    in_specs=[
        _row_spec(IN_CH),
        pl.BlockSpec((IN_CH, IN_CH), lambda i: (0, 0)),
        pl.BlockSpec((NC, BR, 16), lambda i: (0, i, 0)),
    ],
    out_specs=[_row_spec(IN_CH), _row_spec(IN_CH)],
    out_shape=[jax.ShapeDtypeStruct((NPAD, IN_CH), jnp.float32)] * 2,
)

_tc2 = pl.pallas_call(
    _tc2_body,
    grid=(GRID,),
    in_specs=[
        pl.BlockSpec((NC, BR, IN_CH), lambda i: (0, i, 0)),
        _row_spec(IN_CH),
        _row_spec(IN_CH),
        pl.BlockSpec((1, IN_CH), lambda i: (0, 0)),
    ],
    out_specs=[_row_spec(IN_CH), pl.BlockSpec((8, IN_CH), lambda i: (0, 0))],
    out_shape=[
        jax.ShapeDtypeStruct((NPAD, IN_CH), jnp.float32),
        jax.ShapeDtypeStruct((8, IN_CH), jnp.float32),
    ],
)

_tc3 = pl.pallas_call(
    _tc3_body,
    grid=(GRID,),
    in_specs=[
        _row_spec(IN_CH),
        pl.BlockSpec((8, IN_CH), lambda i: (0, 0)),
        pl.BlockSpec((1, IN_CH), lambda i: (0, 0)),
        pl.BlockSpec((1, IN_CH), lambda i: (0, 0)),
        pl.BlockSpec((IN_CH, LAT), lambda i: (0, 0)),
        _row_spec(IN_CH),
    ],
    out_specs=_row_spec(LAT),
    out_shape=jax.ShapeDtypeStruct((NPAD, LAT), jnp.float32),
)

_tc4 = pl.pallas_call(
    _tc4_body,
    grid=(GRID,),
    in_specs=[
        pl.BlockSpec((NC, BR, LAT), lambda i: (0, i, 0)),
        _row_spec(LAT),
        _row_spec(IN_CH),
        pl.BlockSpec((1, LAT), lambda i: (0, 0)),
    ],
    out_specs=_row_spec(LAT),
    out_shape=jax.ShapeDtypeStruct((NPAD, LAT), jnp.float32),
)


def kernel(x, edge_index, W1, b1, gamma, beta, W2, b2):
    src = edge_index[0].astype(jnp.int32)
    dst = edge_index[1].astype(jnp.int32)
    padi = jnp.full((EPAD - E,), NPAD - 1, jnp.int32)
    srcp = jnp.concatenate([src, padi])
    dstp = jnp.concatenate([dst, padi])
    xpad = jnp.concatenate([x, jnp.zeros((NPAD - N, IN_CH), jnp.float32)])

    degp = _sc_degree(dstp)
    scaled1, dinvb = _tc1(xpad, W1, degp)
    part1 = _sc_msg128(scaled1, srcp, dstp)
    out1, stats = _tc2(part1, scaled1, dinvb, b1.reshape(1, IN_CH))
    scaled2 = _tc3(
        out1, stats, gamma.reshape(1, IN_CH), beta.reshape(1, IN_CH), W2, dinvb
    )
    part2 = _sc_msg64(scaled2, srcp, dstp)
    z = _tc4(part2, scaled2, dinvb, b2.reshape(1, LAT))
    return z[:N]
